# baseline (speedup 1.0000x reference)
import functools

import jax
import jax.numpy as jnp
from jax import lax
from jax.experimental import pallas as pl
from jax.experimental.pallas import tpu as pltpu
from jax.experimental.pallas import tpu_sc as plsc

_NCORE = 2
_NSUB = 16
_NW = _NCORE * _NSUB
_LANES = 16
_CHUNK = 4000


def _build_sc_call(n_nodes, n_feat, n_chunks, n_layers):
    cols_pw = n_feat // _NW
    n_pairs = cols_pw // 2
    flat = cols_pw * n_nodes
    n_grp = _CHUNK // _LANES
    n_ngrp = n_nodes // _LANES

    def body(xt_hbm, ed_hbm, out_hbm, hw, acc, hp, eb0, eb1, se0, se1):
        wid = lax.axis_index("c") * _NSUB + lax.axis_index("s")
        pltpu.sync_copy(xt_hbm.at[wid], hw)
        pltpu.sync_copy(xt_hbm.at[wid], acc)

        zeros16 = jnp.zeros((_LANES,), jnp.float32)

        def pack_zero(i, add_acc, zero):
            o = i * _LANES
            vals = []
            for c in range(cols_pw):
                oc = c * n_nodes + o
                v = hw[pl.ds(oc, _LANES)]
                vals.append(v)
                if add_acc:
                    acc[pl.ds(oc, _LANES)] = acc[pl.ds(oc, _LANES)] + v
            for p in range(n_pairs):
                pk = plsc.pack(vals[2 * p], vals[2 * p + 1],
                               format=plsc.PackFormat.INTERLEAVED)
                hp[pl.ds(p * n_nodes + o, _LANES)] = plsc.bitcast(pk, jnp.int32)
            if zero:
                for c in range(cols_pw):
                    hw[pl.ds(c * n_nodes + o, _LANES)] = zeros16

        @plsc.parallel_loop(0, n_ngrp, unroll=2)
        def _init(i):
            pack_zero(i, add_acc=False, zero=True)

        stag = (wid * max(1, n_chunks // _NW)) % n_chunks

        bufs = ((eb0, se0), (eb1, se1))

        def start(ci, b):
            eb, se = bufs[b]
            ch = lax.rem(ci + stag, n_chunks)
            pltpu.async_copy(ed_hbm.at[ch], eb, se)

        def wait(b):
            eb, se = bufs[b]
            pltpu.make_async_copy(ed_hbm.at[0], eb, se).wait()

        def compute(b, target):
            eb, _ = bufs[b]

            @plsc.parallel_loop(0, n_grp, unroll=4)
            def _grp(g):
                o = g * _LANES
                packed = eb[pl.ds(o, _LANES)]
                srcv = packed & 0xFFFF
                dstv = lax.shift_right_logical(packed, 16)
                wv = plsc.bitcast(eb[pl.ds(_CHUNK + o, _LANES)], jnp.float32)
                for p in range(n_pairs):
                    ga = srcv + (p * n_nodes) if p else srcv
                    pk = plsc.load_gather(hp, [ga])
                    a, bv = plsc.unpack(plsc.bitcast(pk, jnp.bfloat16),
                                        format=plsc.PackFormat.INTERLEAVED)
                    c0 = 2 * p
                    sa = dstv + (c0 * n_nodes) if c0 else dstv
                    plsc.addupdate_scatter(target, [sa], a * wv)
                    plsc.addupdate_scatter(target, [dstv + (c0 + 1) * n_nodes],
                                           bv * wv)

        start(0, 0)
        for l in range(n_layers):
            target = hw if l + 1 < n_layers else acc

            def pair(p, carry, target=target):
                i0 = p * 2
                wait(0)
                start(i0 + 1, 1)
                compute(0, target)
                wait(1)
                start(i0 + 2, 0)
                compute(1, target)
                return carry

            lax.fori_loop(0, n_chunks // 2, pair, 0)
            if l + 1 < n_layers:
                @plsc.parallel_loop(0, n_ngrp, unroll=2)
                def _mz(i, zero=(l + 2 < n_layers)):
                    pack_zero(i, add_acc=True, zero=zero)

        wait(0)
        pltpu.sync_copy(acc, out_hbm.at[wid])

    return pl.kernel(
        body,
        out_type=jax.ShapeDtypeStruct((_NW, flat), jnp.float32),
        mesh=plsc.VectorSubcoreMesh(core_axis_name="c", subcore_axis_name="s"),
        compiler_params=pltpu.CompilerParams(needs_layout_passes=False),
        scratch_types=[
            pltpu.VMEM((flat,), jnp.float32),
            pltpu.VMEM((flat,), jnp.float32),
            pltpu.VMEM(((flat // 2),), jnp.int32),
            pltpu.VMEM((2 * _CHUNK,), jnp.int32),
            pltpu.VMEM((2 * _CHUNK,), jnp.int32),
            pltpu.SemaphoreType.DMA,
            pltpu.SemaphoreType.DMA,
        ],
    )


def kernel(x, edge_index, edge_weight):
    n_nodes, n_feat = x.shape
    n_edges = edge_weight.shape[0]
    n_layers = 3

    src = edge_index[0]
    dst = edge_index[1]
    w = edge_weight
    pad = (-n_edges) % (2 * _CHUNK)
    if pad:
        src = jnp.concatenate([src, jnp.zeros((pad,), src.dtype)])
        dst = jnp.concatenate(
            [dst, (jnp.arange(pad, dtype=dst.dtype) % n_nodes)])
        w = jnp.concatenate([w, jnp.zeros((pad,), w.dtype)])
    n_chunks = (n_edges + pad) // _CHUNK

    packed = src | (dst << 16)
    wbits = lax.bitcast_convert_type(w, jnp.int32)
    ed = jnp.concatenate(
        [packed.reshape(n_chunks, _CHUNK), wbits.reshape(n_chunks, _CHUNK)],
        axis=1,
    )
    xt = x.T.reshape(_NW, (n_feat // _NW) * n_nodes)

    call = _build_sc_call(n_nodes, n_feat, n_chunks, n_layers)
    out_t = call(xt, ed)
    return out_t.reshape(n_feat, n_nodes).T


# speedup vs baseline: 1.0191x; 1.0191x over previous
import functools

import jax
import jax.numpy as jnp
from jax import lax
from jax.experimental import pallas as pl
from jax.experimental.pallas import tpu as pltpu
from jax.experimental.pallas import tpu_sc as plsc

_NCORE = 2
_NSUB = 16
_NW = _NCORE * _NSUB
_LANES = 16
_CHUNK = 3200


def _build_sc_call(n_nodes, n_feat, n_chunks, n_layers):
    cols_pw = n_feat // _NW
    n_pairs = cols_pw // 2
    flat = cols_pw * n_nodes
    n_grp = _CHUNK // _LANES
    n_ngrp = n_nodes // _LANES

    def body(xt_hbm, ed_hbm, out_hbm, hw, acc, hp, eb0, eb1, se0, se1):
        wid = lax.axis_index("c") * _NSUB + lax.axis_index("s")
        pltpu.sync_copy(xt_hbm.at[wid], hw)
        pltpu.sync_copy(xt_hbm.at[wid], acc)

        zeros16 = jnp.zeros((_LANES,), jnp.float32)

        def pack_zero(i, add_acc, zero):
            o = i * _LANES
            vals = []
            for c in range(cols_pw):
                oc = c * n_nodes + o
                v = hw[pl.ds(oc, _LANES)]
                vals.append(v)
                if add_acc:
                    acc[pl.ds(oc, _LANES)] = acc[pl.ds(oc, _LANES)] + v
            for p in range(n_pairs):
                pk = plsc.pack(vals[2 * p], vals[2 * p + 1],
                               format=plsc.PackFormat.INTERLEAVED)
                hp[pl.ds(p * n_nodes + o, _LANES)] = plsc.bitcast(pk, jnp.int32)
            if zero:
                for c in range(cols_pw):
                    hw[pl.ds(c * n_nodes + o, _LANES)] = zeros16

        @plsc.parallel_loop(0, n_ngrp, unroll=2)
        def _init(i):
            pack_zero(i, add_acc=False, zero=True)

        stag = (wid * max(1, n_chunks // _NW)) % n_chunks

        bufs = ((eb0, se0), (eb1, se1))

        def start(ci, b):
            eb, se = bufs[b]
            ch = lax.rem(ci + stag, n_chunks)
            pltpu.async_copy(ed_hbm.at[ch], eb, se)

        def wait(b):
            eb, se = bufs[b]
            pltpu.make_async_copy(ed_hbm.at[0], eb, se).wait()

        def compute(b, target):
            eb, _ = bufs[b]

            @plsc.parallel_loop(0, n_grp, unroll=4)
            def _grp(g):
                o = g * _LANES
                packed = eb[pl.ds(o, _LANES)]
                srcv = packed & 0xFFFF
                dstv = lax.shift_right_logical(packed, 16)
                wv = plsc.bitcast(eb[pl.ds(_CHUNK + o, _LANES)], jnp.float32)
                for p in range(n_pairs):
                    ga = srcv + (p * n_nodes) if p else srcv
                    pk = plsc.load_gather(hp, [ga])
                    a, bv = plsc.unpack(plsc.bitcast(pk, jnp.bfloat16),
                                        format=plsc.PackFormat.INTERLEAVED)
                    c0 = 2 * p
                    sa = dstv + (c0 * n_nodes) if c0 else dstv
                    plsc.addupdate_scatter(target, [sa], a * wv)
                    plsc.addupdate_scatter(target, [dstv + (c0 + 1) * n_nodes],
                                           bv * wv)

        start(0, 0)
        for l in range(n_layers):
            target = hw if l + 1 < n_layers else acc

            def pair(p, carry, target=target):
                i0 = p * 2
                wait(0)
                start(i0 + 1, 1)
                compute(0, target)
                wait(1)
                start(i0 + 2, 0)
                compute(1, target)
                return carry

            lax.fori_loop(0, n_chunks // 2, pair, 0)
            if l + 1 < n_layers:
                @plsc.parallel_loop(0, n_ngrp, unroll=2)
                def _mz(i, zero=(l + 2 < n_layers)):
                    pack_zero(i, add_acc=True, zero=zero)

        wait(0)
        pltpu.sync_copy(acc, out_hbm.at[wid])

    return pl.kernel(
        body,
        out_type=jax.ShapeDtypeStruct((_NW, flat), jnp.float32),
        mesh=plsc.VectorSubcoreMesh(core_axis_name="c", subcore_axis_name="s"),
        compiler_params=pltpu.CompilerParams(needs_layout_passes=False),
        scratch_types=[
            pltpu.VMEM((flat,), jnp.float32),
            pltpu.VMEM((flat,), jnp.float32),
            pltpu.VMEM(((flat // 2),), jnp.int32),
            pltpu.VMEM((2 * _CHUNK,), jnp.int32),
            pltpu.VMEM((2 * _CHUNK,), jnp.int32),
            pltpu.SemaphoreType.DMA,
            pltpu.SemaphoreType.DMA,
        ],
    )


def kernel(x, edge_index, edge_weight):
    n_nodes, n_feat = x.shape
    n_edges = edge_weight.shape[0]
    n_layers = 3

    src = edge_index[0]
    dst = edge_index[1]
    w = edge_weight
    pad = (-n_edges) % (2 * _CHUNK)
    if pad:
        src = jnp.concatenate([src, jnp.zeros((pad,), src.dtype)])
        dst = jnp.concatenate(
            [dst, (jnp.arange(pad, dtype=dst.dtype) % n_nodes)])
        w = jnp.concatenate([w, jnp.zeros((pad,), w.dtype)])
    n_chunks = (n_edges + pad) // _CHUNK

    packed = src | (dst << 16)
    wbits = lax.bitcast_convert_type(w, jnp.int32)
    ed = jnp.concatenate(
        [packed.reshape(n_chunks, _CHUNK), wbits.reshape(n_chunks, _CHUNK)],
        axis=1,
    )
    xt = x.T.reshape(_NW, (n_feat // _NW) * n_nodes)

    call = _build_sc_call(n_nodes, n_feat, n_chunks, n_layers)
    out_t = call(xt, ed)
    return out_t.reshape(n_feat, n_nodes).T


# speedup vs baseline: 1.0219x; 1.0028x over previous
import functools

import jax
import jax.numpy as jnp
from jax import lax
from jax.experimental import pallas as pl
from jax.experimental.pallas import tpu as pltpu
from jax.experimental.pallas import tpu_sc as plsc

_NCORE = 2
_NSUB = 16
_NW = _NCORE * _NSUB
_LANES = 16
_CHUNK = 3200


def _build_sc_call(n_nodes, n_feat, n_chunks, n_layers):
    cols_pw = n_feat // _NW
    n_pairs = cols_pw // 2
    flat = cols_pw * n_nodes
    n_grp = _CHUNK // _LANES
    n_ngrp = n_nodes // _LANES

    def body(xt_hbm, ed_hbm, out_hbm, hwa, hwb, acc, hp, eb0, eb1, se0, se1):
        wid = lax.axis_index("c") * _NSUB + lax.axis_index("s")
        pltpu.sync_copy(xt_hbm.at[wid], acc)

        zeros16 = jnp.zeros((_LANES,), jnp.float32)

        def pack_zero(i, add_acc, zero):
            o = i * _LANES
            vals = []
            for c in range(cols_pw):
                oc = c * n_nodes + o
                if add_acc:
                    ref = hwa if c % 2 == 0 else hwb
                    v = ref[pl.ds((c // 2) * n_nodes + o, _LANES)]
                    acc[pl.ds(oc, _LANES)] = acc[pl.ds(oc, _LANES)] + v
                else:
                    v = acc[pl.ds(oc, _LANES)]
                vals.append(v)
            for p in range(n_pairs):
                pk = plsc.pack(vals[2 * p], vals[2 * p + 1],
                               format=plsc.PackFormat.INTERLEAVED)
                hp[pl.ds(p * n_nodes + o, _LANES)] = plsc.bitcast(pk, jnp.int32)
            if zero:
                for p in range(n_pairs):
                    hwa[pl.ds(p * n_nodes + o, _LANES)] = zeros16
                    hwb[pl.ds(p * n_nodes + o, _LANES)] = zeros16

        @plsc.parallel_loop(0, n_ngrp, unroll=2)
        def _init(i):
            pack_zero(i, add_acc=False, zero=True)

        stag = (wid * max(1, n_chunks // _NW)) % n_chunks

        bufs = ((eb0, se0), (eb1, se1))

        def start(ci, b):
            eb, se = bufs[b]
            ch = lax.rem(ci + stag, n_chunks)
            pltpu.async_copy(ed_hbm.at[ch], eb, se)

        def wait(b):
            eb, se = bufs[b]
            pltpu.make_async_copy(ed_hbm.at[0], eb, se).wait()

        def compute(b, target):
            eb, _ = bufs[b]

            @plsc.parallel_loop(0, n_grp, unroll=4)
            def _grp(g):
                o = g * _LANES
                packed = eb[pl.ds(o, _LANES)]
                srcv = packed & 0xFFFF
                dstv = lax.shift_right_logical(packed, 16)
                wv = plsc.bitcast(eb[pl.ds(_CHUNK + o, _LANES)], jnp.float32)
                for p in range(n_pairs):
                    ga = srcv + (p * n_nodes) if p else srcv
                    pk = plsc.load_gather(hp, [ga])
                    a, bv = plsc.unpack(plsc.bitcast(pk, jnp.bfloat16),
                                        format=plsc.PackFormat.INTERLEAVED)
                    if target is None:
                        sa = dstv + (p * n_nodes) if p else dstv
                        plsc.addupdate_scatter(hwa, [sa], a * wv)
                        plsc.addupdate_scatter(hwb, [sa], bv * wv)
                    else:
                        c0 = 2 * p
                        sa = dstv + (c0 * n_nodes) if c0 else dstv
                        plsc.addupdate_scatter(target, [sa], a * wv)
                        plsc.addupdate_scatter(
                            target, [dstv + (c0 + 1) * n_nodes], bv * wv)

        start(0, 0)
        for l in range(n_layers):
            target = None if l + 1 < n_layers else acc

            def pair(p, carry, target=target):
                i0 = p * 2
                wait(0)
                start(i0 + 1, 1)
                compute(0, target)
                wait(1)
                start(i0 + 2, 0)
                compute(1, target)
                return carry

            lax.fori_loop(0, n_chunks // 2, pair, 0)
            if l + 1 < n_layers:
                @plsc.parallel_loop(0, n_ngrp, unroll=2)
                def _mz(i, zero=(l + 2 < n_layers)):
                    pack_zero(i, add_acc=True, zero=zero)

        wait(0)
        pltpu.sync_copy(acc, out_hbm.at[wid])

    return pl.kernel(
        body,
        out_type=jax.ShapeDtypeStruct((_NW, flat), jnp.float32),
        mesh=plsc.VectorSubcoreMesh(core_axis_name="c", subcore_axis_name="s"),
        compiler_params=pltpu.CompilerParams(needs_layout_passes=False),
        scratch_types=[
            pltpu.VMEM((flat // 2,), jnp.float32),
            pltpu.VMEM((flat // 2,), jnp.float32),
            pltpu.VMEM((flat,), jnp.float32),
            pltpu.VMEM(((flat // 2),), jnp.int32),
            pltpu.VMEM((2 * _CHUNK,), jnp.int32),
            pltpu.VMEM((2 * _CHUNK,), jnp.int32),
            pltpu.SemaphoreType.DMA,
            pltpu.SemaphoreType.DMA,
        ],
    )


def kernel(x, edge_index, edge_weight):
    n_nodes, n_feat = x.shape
    n_edges = edge_weight.shape[0]
    n_layers = 3

    src = edge_index[0]
    dst = edge_index[1]
    w = edge_weight
    pad = (-n_edges) % (2 * _CHUNK)
    if pad:
        src = jnp.concatenate([src, jnp.zeros((pad,), src.dtype)])
        dst = jnp.concatenate(
            [dst, (jnp.arange(pad, dtype=dst.dtype) % n_nodes)])
        w = jnp.concatenate([w, jnp.zeros((pad,), w.dtype)])
    n_chunks = (n_edges + pad) // _CHUNK

    packed = src | (dst << 16)
    wbits = lax.bitcast_convert_type(w, jnp.int32)
    ed = jnp.concatenate(
        [packed.reshape(n_chunks, _CHUNK), wbits.reshape(n_chunks, _CHUNK)],
        axis=1,
    )
    xt = x.T.reshape(_NW, (n_feat // _NW) * n_nodes)

    call = _build_sc_call(n_nodes, n_feat, n_chunks, n_layers)
    out_t = call(xt, ed)
    return out_t.reshape(n_feat, n_nodes).T
